# baseline (speedup 1.0000x reference)
import jax
import jax.numpy as jnp
from jax import lax
from jax.experimental import pallas as pl
from jax.experimental.pallas import tpu as pltpu
from jax.experimental.pallas import tpu_sc as plsc

N = 10000
E = 160000
D = 256
DH = 128
RW = 0.001

NC = 2
NS = 16
HALF = N // NC
ROWS_PER_TILE = 320
PAD = NS * ROWS_PER_TILE
TRASH = HALF
G = 128
STEPS = 79
EPTP = STEPS * G
SB = 64


def _sc_body(x2_hbm, src_hbm, dst_hbm, out_hbm, src_v, dst_v, rows_v,
             stage_v, sem0, sem1, sem2, agg_sh):
  c = lax.axis_index("c")
  s = lax.axis_index("s")

  zero16 = jnp.zeros((16,), jnp.float32)

  def zrow(i, _):
    def zcol(j, _):
      stage_v[i, pl.ds(j * 16, 16)] = zero16
      return 0
    return lax.fori_loop(0, DH // 16, zcol, 0)

  lax.fori_loop(0, SB, zrow, 0)

  pltpu.sync_copy(dst_hbm.at[c, s], dst_v)

  for p in range(2):
    pltpu.sync_copy(src_hbm.at[p, s], src_v)

    def zs(q, _):
      pltpu.sync_copy(stage_v,
                      agg_sh.at[pl.ds(s * ROWS_PER_TILE + q * SB, SB)])
      return 0

    lax.fori_loop(0, ROWS_PER_TILE // SB, zs, 0)
    plsc.subcore_barrier()

    pltpu.async_copy(x2_hbm.at[src_v.at[0]], rows_v.at[0], sem0)
    pltpu.async_copy(x2_hbm.at[src_v.at[1]], rows_v.at[1], sem1)

    def step(j, _):
      b = lax.rem(j, 3)

      @pl.when(b == 0)
      def _():
        pltpu.make_async_copy(x2_hbm.at[src_v.at[j]], rows_v.at[0],
                              sem0).wait()
        pltpu.sync_copy(rows_v.at[0], agg_sh.at[dst_v.at[j]], add=True)

      @pl.when(b == 1)
      def _():
        pltpu.make_async_copy(x2_hbm.at[src_v.at[j]], rows_v.at[1],
                              sem1).wait()
        pltpu.sync_copy(rows_v.at[1], agg_sh.at[dst_v.at[j]], add=True)

      @pl.when(b == 2)
      def _():
        pltpu.make_async_copy(x2_hbm.at[src_v.at[j]], rows_v.at[2],
                              sem2).wait()
        pltpu.sync_copy(rows_v.at[2], agg_sh.at[dst_v.at[j]], add=True)

      nxt = j + 2
      bn = lax.rem(nxt, 3)

      @pl.when((nxt < STEPS) & (bn == 0))
      def _():
        pltpu.async_copy(x2_hbm.at[src_v.at[nxt]], rows_v.at[0], sem0)

      @pl.when((nxt < STEPS) & (bn == 1))
      def _():
        pltpu.async_copy(x2_hbm.at[src_v.at[nxt]], rows_v.at[1], sem1)

      @pl.when((nxt < STEPS) & (bn == 2))
      def _():
        pltpu.async_copy(x2_hbm.at[src_v.at[nxt]], rows_v.at[2], sem2)

      return 0

    lax.fori_loop(0, STEPS, step, 0)
    plsc.subcore_barrier()

    def co(q, _):
      pltpu.sync_copy(agg_sh.at[pl.ds(s * ROWS_PER_TILE + q * SB, SB)],
                      stage_v)
      pltpu.sync_copy(stage_v,
                      out_hbm.at[c, p, pl.ds(s * ROWS_PER_TILE + q * SB, SB)])
      return 0

    lax.fori_loop(0, ROWS_PER_TILE // SB, co, 0)
    if p == 0:
      lax.fori_loop(0, SB, zrow, 0)


@jax.jit
def _segment_sum_sc(x2, src4, dst4):
  mesh = plsc.VectorSubcoreMesh(core_axis_name="c", subcore_axis_name="s",
                                num_cores=NC, num_subcores=NS)
  f = pl.kernel(
      _sc_body,
      out_type=jax.ShapeDtypeStruct((NC, 2, PAD, DH), jnp.float32),
      mesh=mesh,
      scratch_types=[
          pltpu.VMEM((STEPS, G), jnp.int32),
          pltpu.VMEM((STEPS, G), jnp.int32),
          pltpu.VMEM((3, G, DH), jnp.float32),
          pltpu.VMEM((SB, DH), jnp.float32),
          pltpu.SemaphoreType.DMA,
          pltpu.SemaphoreType.DMA,
          pltpu.SemaphoreType.DMA,
          pltpu.VMEM_SHARED((PAD, DH), jnp.float32),
      ],
  )
  return f(x2, src4, dst4)


def _tc_body(x_ref, agg_ref, wl1, wr1, wres1, wl2, wr2, wres2,
             bl1, bres1, bl2, bres2, loc_ref, scale_ref):
  ap = agg_ref[...]
  a1 = RW * wl1[...]
  b1 = RW * wr1[...] + (1.0 - RW) * wres1[...]
  a2 = RW * wl2[...]
  b2 = RW * wr2[...] + (1.0 - RW) * wres2[...]
  c1 = RW * bl1[...] + (1.0 - RW) * bres1[...]
  c2 = RW * bl2[...] + (1.0 - RW) * bres2[...]
  hs = jnp.concatenate([ap[0, 0], ap[0, 1], x_ref[...]], axis=1)
  wcat = jnp.concatenate(
      [jnp.concatenate([a1, b1], axis=1),
       jnp.concatenate([a2, b2], axis=1)], axis=0)
  hall = lax.dot_general(hs, wcat, (((1,), (1,)), ((), ())),
                         preferred_element_type=jnp.float32)
  h1 = hall[:, :D] + c1
  h2 = hall[:, D:] + c2
  loc_ref[...] = jnp.clip(h1, -100.0, 100.0)
  scale_ref[...] = jnp.minimum(jax.nn.softplus(h2) + 0.001, 100.0)


@jax.jit
def _dense_tc(x, agg, W_l1, W_r1, W_res1, W_l2, W_r2, W_res2,
              b_l1, b_res1, b_l2, b_res2):
  bm = 1000
  grid = (N // bm,)
  nb = HALF // bm
  row = pl.BlockSpec((bm, D), lambda i: (i, 0))
  rowp = pl.BlockSpec((1, 2, bm, DH), lambda i: (i // nb, 0, i % nb, 0))
  full = pl.BlockSpec((D, D), lambda i: (0, 0))
  vec = pl.BlockSpec((1, D), lambda i: (0, 0))
  return pl.pallas_call(
      _tc_body,
      grid=grid,
      in_specs=[row, rowp, full, full, full, full, full, full,
                vec, vec, vec, vec],
      out_specs=[row, row],
      out_shape=[jax.ShapeDtypeStruct((N, D), jnp.float32),
                 jax.ShapeDtypeStruct((N, D), jnp.float32)],
  )(x, agg, W_l1, W_r1, W_res1, W_l2, W_r2, W_res2,
    b_l1.reshape(1, D), b_res1.reshape(1, D),
    b_l2.reshape(1, D), b_res2.reshape(1, D))


def kernel(x, edge_index, W_l1, b_l1, W_r1, W_res1, b_res1,
           W_l2, b_l2, W_r2, W_res2, b_res2):
  pad_e = NS * EPTP - E
  src = jnp.concatenate([edge_index[0], jnp.zeros((pad_e,), jnp.int32)])
  dst = jnp.concatenate([edge_index[1], jnp.full((pad_e,), N, jnp.int32)])
  src4 = jnp.stack([2 * src, 2 * src + 1]).reshape(2, NS, STEPS, G)
  dst_c0 = jnp.where(dst < HALF, dst, TRASH)
  dst_c1 = jnp.where(dst >= HALF, dst - HALF, TRASH)
  dst4 = jnp.stack([dst_c0, dst_c1]).reshape(NC, NS, STEPS, G)
  x2 = x.reshape(2 * N, DH)

  agg_pad = _segment_sum_sc(x2, src4, dst4)

  loc, scale = _dense_tc(x, agg_pad, W_l1, W_r1, W_res1, W_l2, W_r2,
                         W_res2, b_l1, b_res1, b_l2, b_res2)
  return (loc, scale)


# speedup vs baseline: 1.7635x; 1.7635x over previous
import jax
import jax.numpy as jnp
from jax import lax
from jax.experimental import pallas as pl
from jax.experimental.pallas import tpu as pltpu
from jax.experimental.pallas import tpu_sc as plsc

N = 10000
E = 160000
D = 256
DH = 128
RW = 0.001

NC = 2
NS = 16
HALF = N // NC
ROWS_PER_TILE = 320
PAD = NS * ROWS_PER_TILE
TRASH = HALF
EPT = E // NS
G = 80
STEPS = EPT // G
SB = 64


def _sc_body(x2_hbm, src_hbm, dst_hbm, out_hbm, src_v, dst_v, rows_v,
             stage_v, sem0, sem1, agg_sh):
  c = lax.axis_index("c")
  s = lax.axis_index("s")

  zero16 = jnp.zeros((16,), jnp.float32)

  def zrow(i, _):
    def zcol(j, _):
      stage_v[i, pl.ds(j * 16, 16)] = zero16
      return 0
    return lax.fori_loop(0, DH // 16, zcol, 0)

  lax.fori_loop(0, SB, zrow, 0)

  pltpu.sync_copy(dst_hbm.at[c, s], dst_v)

  for p in range(2):
    pltpu.sync_copy(src_hbm.at[p, s], src_v)
    def zs(q, _):
      pltpu.sync_copy(stage_v,
                      agg_sh.at[pl.ds(s * ROWS_PER_TILE + q * SB, SB)])
      return 0

    lax.fori_loop(0, ROWS_PER_TILE // SB, zs, 0)
    plsc.subcore_barrier()

    pltpu.async_copy(x2_hbm.at[src_v.at[0]], rows_v.at[0], sem0)

    def step(j, _):
      even = lax.rem(j, 2) == 0

      @pl.when((j + 1 < STEPS) & even)
      def _():
        pltpu.async_copy(x2_hbm.at[src_v.at[j + 1]], rows_v.at[1],
                         sem1)

      @pl.when((j + 1 < STEPS) & jnp.logical_not(even))
      def _():
        pltpu.async_copy(x2_hbm.at[src_v.at[j + 1]], rows_v.at[0],
                         sem0)

      @pl.when(even)
      def _():
        pltpu.make_async_copy(x2_hbm.at[src_v.at[j]], rows_v.at[0],
                              sem0).wait()
        pltpu.sync_copy(rows_v.at[0], agg_sh.at[dst_v.at[j]], add=True)

      @pl.when(jnp.logical_not(even))
      def _():
        pltpu.make_async_copy(x2_hbm.at[src_v.at[j]], rows_v.at[1],
                              sem1).wait()
        pltpu.sync_copy(rows_v.at[1], agg_sh.at[dst_v.at[j]], add=True)

      return 0

    lax.fori_loop(0, STEPS, step, 0)
    plsc.subcore_barrier()

    def co(q, _):
      pltpu.sync_copy(agg_sh.at[pl.ds(s * ROWS_PER_TILE + q * SB, SB)],
                      stage_v)
      pltpu.sync_copy(stage_v,
                      out_hbm.at[c, p, pl.ds(s * ROWS_PER_TILE + q * SB, SB)])
      return 0

    lax.fori_loop(0, ROWS_PER_TILE // SB, co, 0)
    if p == 0:
      lax.fori_loop(0, SB, zrow, 0)


@jax.jit
def _segment_sum_sc(x2, src4, dst4):
  mesh = plsc.VectorSubcoreMesh(core_axis_name="c", subcore_axis_name="s",
                                num_cores=NC, num_subcores=NS)
  f = pl.kernel(
      _sc_body,
      out_type=jax.ShapeDtypeStruct((NC, 2, PAD, DH), jnp.float32),
      mesh=mesh,
      scratch_types=[
          pltpu.VMEM((STEPS, G), jnp.int32),
          pltpu.VMEM((STEPS, G), jnp.int32),
          pltpu.VMEM((2, G, DH), jnp.float32),
          pltpu.VMEM((SB, DH), jnp.float32),
          pltpu.SemaphoreType.DMA,
          pltpu.SemaphoreType.DMA,
          pltpu.VMEM_SHARED((PAD, DH), jnp.float32),
      ],
  )
  return f(x2, src4, dst4)


def _tc_body(x_ref, agg_ref, wl1, wr1, wres1, wl2, wr2, wres2,
             bl1, bres1, bl2, bres2, loc_ref, scale_ref):
  ap = agg_ref[...]
  a1 = RW * wl1[...]
  b1 = RW * wr1[...] + (1.0 - RW) * wres1[...]
  a2 = RW * wl2[...]
  b2 = RW * wr2[...] + (1.0 - RW) * wres2[...]
  c1 = RW * bl1[...] + (1.0 - RW) * bres1[...]
  c2 = RW * bl2[...] + (1.0 - RW) * bres2[...]
  hs = jnp.concatenate([ap[0, 0], ap[0, 1], x_ref[...]], axis=1)
  wcat = jnp.concatenate(
      [jnp.concatenate([a1, b1], axis=1),
       jnp.concatenate([a2, b2], axis=1)], axis=0)
  hall = lax.dot_general(hs, wcat, (((1,), (1,)), ((), ())),
                         preferred_element_type=jnp.float32)
  h1 = hall[:, :D] + c1
  h2 = hall[:, D:] + c2
  loc_ref[...] = jnp.clip(h1, -100.0, 100.0)
  scale_ref[...] = jnp.minimum(jax.nn.softplus(h2) + 0.001, 100.0)


@jax.jit
def _dense_tc(x, agg, W_l1, W_r1, W_res1, W_l2, W_r2, W_res2,
              b_l1, b_res1, b_l2, b_res2):
  bm = 1000
  grid = (N // bm,)
  nb = HALF // bm
  row = pl.BlockSpec((bm, D), lambda i: (i, 0))
  rowp = pl.BlockSpec((1, 2, bm, DH), lambda i: (i // nb, 0, i % nb, 0))
  full = pl.BlockSpec((D, D), lambda i: (0, 0))
  vec = pl.BlockSpec((1, D), lambda i: (0, 0))
  return pl.pallas_call(
      _tc_body,
      grid=grid,
      in_specs=[row, rowp, full, full, full, full, full, full,
                vec, vec, vec, vec],
      out_specs=[row, row],
      out_shape=[jax.ShapeDtypeStruct((N, D), jnp.float32),
                 jax.ShapeDtypeStruct((N, D), jnp.float32)],
  )(x, agg, W_l1, W_r1, W_res1, W_l2, W_r2, W_res2,
    b_l1.reshape(1, D), b_res1.reshape(1, D),
    b_l2.reshape(1, D), b_res2.reshape(1, D))


def kernel(x, edge_index, W_l1, b_l1, W_r1, W_res1, b_res1,
           W_l2, b_l2, W_r2, W_res2, b_res2):
  src = edge_index[0]
  dst = edge_index[1]
  src4 = jnp.stack([2 * src, 2 * src + 1]).reshape(2, NS, STEPS, G)
  dst_c0 = jnp.where(dst < HALF, dst, TRASH)
  dst_c1 = jnp.where(dst >= HALF, dst - HALF, TRASH)
  dst4 = jnp.stack([dst_c0, dst_c1]).reshape(NC, NS, STEPS, G)
  x2 = x.reshape(2 * N, DH)

  agg_pad = _segment_sum_sc(x2, src4, dst4)

  loc, scale = _dense_tc(x, agg_pad, W_l1, W_r1, W_res1, W_l2, W_r2,
                         W_res2, b_l1, b_res1, b_l2, b_res2)
  return (loc, scale)
